# Initial kernel scaffold; baseline (speedup 1.0000x reference)
import jax
import jax.numpy as jnp
from jax.experimental import pallas as pl


def kernel(x, sadj, fadj, W1_s1, b1_s1, W2_s1, b2_s1, W1_s2, b1_s2, W2_s2, b2_s2, W1_c, b1_c, W2_c, b2_c, Wp1, bp1, Wp2, Wm, bm):
    raise NotImplementedError("write your pallas kernel here")


# speedup vs baseline: 1.8623x; 1.8623x over previous
import jax
import jax.numpy as jnp
from jax.experimental import pallas as pl
from jax.experimental.pallas import tpu as pltpu

_N = 10000
_F = 128
_H1 = 64
_H2 = 32
_BR = 200
_BE = 2000
_NBUF = 3

_DN = (((1,), (0,)), ((), ()))


def _proj1_body(x_ref, ws_ref, wf_ref, ss_ref, sf_ref):
    x = x_ref[...]
    ss_ref[...] = jax.lax.dot_general(
        x, ws_ref[...], _DN,
        preferred_element_type=jnp.float32).astype(jnp.bfloat16)
    sf_ref[...] = jax.lax.dot_general(
        x, wf_ref[...], _DN,
        preferred_element_type=jnp.float32).astype(jnp.bfloat16)


def _proj1(x, w1s, w1f):
    return pl.pallas_call(
        _proj1_body,
        grid=(_N // _BE,),
        in_specs=[
            pl.BlockSpec((_BE, _F), lambda i: (i, 0)),
            pl.BlockSpec((_F, _H1), lambda i: (0, 0)),
            pl.BlockSpec((_F, _H1), lambda i: (0, 0)),
        ],
        out_specs=[
            pl.BlockSpec((_BE, _H1), lambda i: (i, 0)),
            pl.BlockSpec((_BE, _H1), lambda i: (i, 0)),
        ],
        out_shape=[
            jax.ShapeDtypeStruct((_N, _H1), jnp.bfloat16),
            jax.ShapeDtypeStruct((_N, _H1), jnp.bfloat16),
        ],
        compiler_params=pltpu.CompilerParams(
            dimension_semantics=("arbitrary",)),
    )(x, w1s, w1f)


def _mega_body(sadj_r, fadj_r, ss_r, sf_r, b1s_r, b1f_r, w2s_r, w2f_r,
               b2s_r, b2f_r, wp1_r, bp1_r, wp2_r, wm_r, bm_r,
               emb1_r, com1_r, com2_r, emb2_r, out_r, beta_r, emb_r,
               ts_h, tf_h, ts_v, tf_v):
    adj_spec = pl.BlockSpec((_BR, _N), lambda i: (i, 0),
                            pipeline_mode=pl.Buffered(buffer_count=_NBUF))
    t_spec = pl.BlockSpec((_BR, _H2), lambda i: (i, 0))
    o16 = pl.BlockSpec((_BR, 16), lambda i: (i, 0))
    o3 = pl.BlockSpec((_BR, 3), lambda i: (i, 0))

    def pass1(sadj_blk, fadj_blk, ts_blk, tf_blk):
        acc_s = jax.lax.dot_general(sadj_blk[...].astype(jnp.bfloat16),
                                    ss_r[...], _DN,
                                    preferred_element_type=jnp.float32)
        h_s = jnp.maximum(acc_s + b1s_r[...], 0.0)
        ts_blk[...] = jax.lax.dot_general(
            h_s, w2s_r[...], _DN,
            preferred_element_type=jnp.float32).astype(jnp.bfloat16)
        acc_f = jax.lax.dot_general(fadj_blk[...].astype(jnp.bfloat16),
                                    sf_r[...], _DN,
                                    preferred_element_type=jnp.float32)
        h_f = jnp.maximum(acc_f + b1f_r[...], 0.0)
        tf_blk[...] = jax.lax.dot_general(
            h_f, w2f_r[...], _DN,
            preferred_element_type=jnp.float32).astype(jnp.bfloat16)

    pltpu.emit_pipeline(
        pass1, grid=(_N // _BR,),
        in_specs=[adj_spec, adj_spec],
        out_specs=[t_spec, t_spec],
    )(sadj_r, fadj_r, ts_h, tf_h)

    pltpu.sync_copy(ts_h, ts_v)
    pltpu.sync_copy(tf_h, tf_v)

    def pass2(sadj_blk, fadj_blk, emb1_b, com1_b, com2_b, emb2_b,
              out_b, beta_b, emb_b):
        o_s = jax.lax.dot_general(sadj_blk[...].astype(jnp.bfloat16),
                                  ts_v[...], _DN,
                                  preferred_element_type=jnp.float32) + b2s_r[...]
        o_f = jax.lax.dot_general(fadj_blk[...].astype(jnp.bfloat16),
                                  tf_v[...], _DN,
                                  preferred_element_type=jnp.float32) + b2f_r[...]
        e1 = o_s[:, :16]
        c1 = o_s[:, 16:]
        e2 = o_f[:, :16]
        c2 = o_f[:, 16:]
        emb1_b[...] = e1
        com1_b[...] = c1
        com2_b[...] = c2
        emb2_b[...] = e2
        xc = 0.5 * (c1 + c2)

        wp1 = wp1_r[...]
        bp1 = bp1_r[...]
        wp2 = wp2_r[...]

        def att_score(e):
            t = jnp.tanh(jax.lax.dot_general(
                e, wp1, _DN, preferred_element_type=jnp.float32) + bp1)
            return jnp.sum(t * wp2, axis=1, keepdims=True)

        w1 = att_score(e1)
        w2 = att_score(e2)
        w3 = att_score(xc)
        m = jnp.maximum(jnp.maximum(w1, w2), w3)
        a1 = jnp.exp(w1 - m)
        a2 = jnp.exp(w2 - m)
        a3 = jnp.exp(w3 - m)
        inv = 1.0 / (a1 + a2 + a3)
        b1 = a1 * inv
        b2 = a2 * inv
        b3 = a3 * inv
        beta_b[:, 0:1] = b1
        beta_b[:, 1:2] = b2
        beta_b[:, 2:3] = b3

        emb = b1 * e1 + b2 * e2 + b3 * xc
        emb_b[...] = emb

        logits = jax.lax.dot_general(
            emb, wm_r[...], _DN,
            preferred_element_type=jnp.float32) + bm_r[...]
        lm = jnp.max(logits, axis=1, keepdims=True)
        lse = jnp.log(jnp.sum(jnp.exp(logits - lm), axis=1, keepdims=True)) + lm
        out_b[...] = logits - lse

    pltpu.emit_pipeline(
        pass2, grid=(_N // _BR,),
        in_specs=[adj_spec, adj_spec],
        out_specs=[o16, o16, o16, o16, o16, o3, o16],
    )(sadj_r, fadj_r, emb1_r, com1_r, com2_r, emb2_r, out_r, beta_r, emb_r)


def _mega(sadj, fadj, s_s, s_f, b1s, b1f, w2s, w2f, b2s, b2f,
          wp1, bp1, wp2, wm, bm):
    vspec = pl.BlockSpec(memory_space=pltpu.VMEM)
    aspec = pl.BlockSpec(memory_space=pl.ANY)
    return pl.pallas_call(
        _mega_body,
        in_specs=[aspec, aspec] + [vspec] * 13,
        out_specs=[aspec] * 9,
        out_shape=[
            jax.ShapeDtypeStruct((_N, 16), jnp.float32),
            jax.ShapeDtypeStruct((_N, 16), jnp.float32),
            jax.ShapeDtypeStruct((_N, 16), jnp.float32),
            jax.ShapeDtypeStruct((_N, 16), jnp.float32),
            jax.ShapeDtypeStruct((_N, 16), jnp.float32),
            jax.ShapeDtypeStruct((_N, 3), jnp.float32),
            jax.ShapeDtypeStruct((_N, 16), jnp.float32),
            jax.ShapeDtypeStruct((_N, _H2), jnp.bfloat16),
            jax.ShapeDtypeStruct((_N, _H2), jnp.bfloat16),
        ],
        scratch_shapes=[
            pltpu.VMEM((_N, _H2), jnp.bfloat16),
            pltpu.VMEM((_N, _H2), jnp.bfloat16),
        ],
    )(sadj, fadj, s_s, s_f, b1s, b1f, w2s, w2f, b2s, b2f,
      wp1, bp1, wp2, wm, bm)


def kernel(x, sadj, fadj, W1_s1, b1_s1, W2_s1, b2_s1, W1_s2, b1_s2, W2_s2,
           b2_s2, W1_c, b1_c, W2_c, b2_c, Wp1, bp1, Wp2, Wm, bm):
    w1s = jnp.concatenate([W1_s1, W1_c], axis=1)
    w1f = jnp.concatenate([W1_s2, W1_c], axis=1)
    b1s = jnp.concatenate([b1_s1, b1_c])[None, :]
    b1f = jnp.concatenate([b1_s2, b1_c])[None, :]
    z = jnp.zeros((32, 16), jnp.float32)
    w2s = jnp.concatenate([jnp.concatenate([W2_s1, z], axis=1),
                           jnp.concatenate([z, W2_c], axis=1)], axis=0)
    w2f = jnp.concatenate([jnp.concatenate([W2_s2, z], axis=1),
                           jnp.concatenate([z, W2_c], axis=1)], axis=0)
    b2s = jnp.concatenate([b2_s1, b2_c])[None, :]
    b2f = jnp.concatenate([b2_s2, b2_c])[None, :]

    s_s, s_f = _proj1(x, w1s, w1f)
    emb1, com1, com2, emb2, output, beta3, emb, _, _ = _mega(
        sadj, fadj, s_s, s_f, b1s, b1f, w2s, w2f, b2s, b2f,
        Wp1, bp1[None, :], Wp2[:, 0][None, :], Wm, bm[None, :])

    return (output, beta3[:, :, None], emb1, com1, com2, emb2, emb)


# speedup vs baseline: 1.8648x; 1.0013x over previous
import jax
import jax.numpy as jnp
from jax.experimental import pallas as pl
from jax.experimental.pallas import tpu as pltpu

_N = 10000
_F = 128
_H1 = 64
_H2 = 32
_BR = 200
_BE = 2000
_NBUF = 3

_DN = (((1,), (0,)), ((), ()))


def _proj1_body(x_ref, ws_ref, wf_ref, ss_ref, sf_ref):
    x = x_ref[...]
    ss_ref[...] = jax.lax.dot_general(
        x, ws_ref[...], _DN,
        preferred_element_type=jnp.float32).astype(jnp.bfloat16)
    sf_ref[...] = jax.lax.dot_general(
        x, wf_ref[...], _DN,
        preferred_element_type=jnp.float32).astype(jnp.bfloat16)


def _proj1(x, w1s, w1f):
    return pl.pallas_call(
        _proj1_body,
        grid=(_N // _BE,),
        in_specs=[
            pl.BlockSpec((_BE, _F), lambda i: (i, 0)),
            pl.BlockSpec((_F, _H1), lambda i: (0, 0)),
            pl.BlockSpec((_F, _H1), lambda i: (0, 0)),
        ],
        out_specs=[
            pl.BlockSpec((_BE, _H1), lambda i: (i, 0)),
            pl.BlockSpec((_BE, _H1), lambda i: (i, 0)),
        ],
        out_shape=[
            jax.ShapeDtypeStruct((_N, _H1), jnp.bfloat16),
            jax.ShapeDtypeStruct((_N, _H1), jnp.bfloat16),
        ],
        compiler_params=pltpu.CompilerParams(
            dimension_semantics=("arbitrary",)),
    )(x, w1s, w1f)


def _mega_body(sadj_r, fadj_r, ss_r, sf_r, b1s_r, b1f_r, w2s_r, w2f_r,
               b2s_r, b2f_r, wp1_r, bp1_r, wp2_r, wm_r, bm_r,
               emb1_r, com1_r, com2_r, emb2_r, out_r, beta_r, emb_r,
               ts_h, tf_h, ts_v, tf_v):
    adj_spec = pl.BlockSpec((_BR, _N), lambda i: (i, 0),
                            pipeline_mode=pl.Buffered(buffer_count=_NBUF, use_lookahead=True))
    t_spec = pl.BlockSpec((_BR, _H2), lambda i: (i, 0))
    o16 = pl.BlockSpec((_BR, 16), lambda i: (i, 0))
    o3 = pl.BlockSpec((_BR, 3), lambda i: (i, 0))

    def pass1(sadj_blk, fadj_blk, ts_blk, tf_blk):
        acc_s = jax.lax.dot_general(sadj_blk[...].astype(jnp.bfloat16),
                                    ss_r[...], _DN,
                                    preferred_element_type=jnp.float32)
        h_s = jnp.maximum(acc_s + b1s_r[...], 0.0)
        ts_blk[...] = jax.lax.dot_general(
            h_s, w2s_r[...], _DN,
            preferred_element_type=jnp.float32).astype(jnp.bfloat16)
        acc_f = jax.lax.dot_general(fadj_blk[...].astype(jnp.bfloat16),
                                    sf_r[...], _DN,
                                    preferred_element_type=jnp.float32)
        h_f = jnp.maximum(acc_f + b1f_r[...], 0.0)
        tf_blk[...] = jax.lax.dot_general(
            h_f, w2f_r[...], _DN,
            preferred_element_type=jnp.float32).astype(jnp.bfloat16)

    pltpu.emit_pipeline(
        pass1, grid=(_N // _BR,),
        in_specs=[adj_spec, adj_spec],
        out_specs=[t_spec, t_spec],
    )(sadj_r, fadj_r, ts_h, tf_h)

    pltpu.sync_copy(ts_h, ts_v)
    pltpu.sync_copy(tf_h, tf_v)

    def pass2(sadj_blk, fadj_blk, emb1_b, com1_b, com2_b, emb2_b,
              out_b, beta_b, emb_b):
        o_s = jax.lax.dot_general(sadj_blk[...].astype(jnp.bfloat16),
                                  ts_v[...], _DN,
                                  preferred_element_type=jnp.float32) + b2s_r[...]
        o_f = jax.lax.dot_general(fadj_blk[...].astype(jnp.bfloat16),
                                  tf_v[...], _DN,
                                  preferred_element_type=jnp.float32) + b2f_r[...]
        e1 = o_s[:, :16]
        c1 = o_s[:, 16:]
        e2 = o_f[:, :16]
        c2 = o_f[:, 16:]
        emb1_b[...] = e1
        com1_b[...] = c1
        com2_b[...] = c2
        emb2_b[...] = e2
        xc = 0.5 * (c1 + c2)

        wp1 = wp1_r[...]
        bp1 = bp1_r[...]
        wp2 = wp2_r[...]

        def att_score(e):
            t = jnp.tanh(jax.lax.dot_general(
                e, wp1, _DN, preferred_element_type=jnp.float32) + bp1)
            return jnp.sum(t * wp2, axis=1, keepdims=True)

        w1 = att_score(e1)
        w2 = att_score(e2)
        w3 = att_score(xc)
        m = jnp.maximum(jnp.maximum(w1, w2), w3)
        a1 = jnp.exp(w1 - m)
        a2 = jnp.exp(w2 - m)
        a3 = jnp.exp(w3 - m)
        inv = 1.0 / (a1 + a2 + a3)
        b1 = a1 * inv
        b2 = a2 * inv
        b3 = a3 * inv
        beta_b[:, 0:1] = b1
        beta_b[:, 1:2] = b2
        beta_b[:, 2:3] = b3

        emb = b1 * e1 + b2 * e2 + b3 * xc
        emb_b[...] = emb

        logits = jax.lax.dot_general(
            emb, wm_r[...], _DN,
            preferred_element_type=jnp.float32) + bm_r[...]
        lm = jnp.max(logits, axis=1, keepdims=True)
        lse = jnp.log(jnp.sum(jnp.exp(logits - lm), axis=1, keepdims=True)) + lm
        out_b[...] = logits - lse

    pltpu.emit_pipeline(
        pass2, grid=(_N // _BR,),
        in_specs=[adj_spec, adj_spec],
        out_specs=[o16, o16, o16, o16, o16, o3, o16],
    )(sadj_r, fadj_r, emb1_r, com1_r, com2_r, emb2_r, out_r, beta_r, emb_r)


def _mega(sadj, fadj, s_s, s_f, b1s, b1f, w2s, w2f, b2s, b2f,
          wp1, bp1, wp2, wm, bm):
    vspec = pl.BlockSpec(memory_space=pltpu.VMEM)
    aspec = pl.BlockSpec(memory_space=pl.ANY)
    return pl.pallas_call(
        _mega_body,
        in_specs=[aspec, aspec] + [vspec] * 13,
        out_specs=[aspec] * 9,
        out_shape=[
            jax.ShapeDtypeStruct((_N, 16), jnp.float32),
            jax.ShapeDtypeStruct((_N, 16), jnp.float32),
            jax.ShapeDtypeStruct((_N, 16), jnp.float32),
            jax.ShapeDtypeStruct((_N, 16), jnp.float32),
            jax.ShapeDtypeStruct((_N, 16), jnp.float32),
            jax.ShapeDtypeStruct((_N, 3), jnp.float32),
            jax.ShapeDtypeStruct((_N, 16), jnp.float32),
            jax.ShapeDtypeStruct((_N, _H2), jnp.bfloat16),
            jax.ShapeDtypeStruct((_N, _H2), jnp.bfloat16),
        ],
        scratch_shapes=[
            pltpu.VMEM((_N, _H2), jnp.bfloat16),
            pltpu.VMEM((_N, _H2), jnp.bfloat16),
        ],
    )(sadj, fadj, s_s, s_f, b1s, b1f, w2s, w2f, b2s, b2f,
      wp1, bp1, wp2, wm, bm)


def kernel(x, sadj, fadj, W1_s1, b1_s1, W2_s1, b2_s1, W1_s2, b1_s2, W2_s2,
           b2_s2, W1_c, b1_c, W2_c, b2_c, Wp1, bp1, Wp2, Wm, bm):
    w1s = jnp.concatenate([W1_s1, W1_c], axis=1)
    w1f = jnp.concatenate([W1_s2, W1_c], axis=1)
    b1s = jnp.concatenate([b1_s1, b1_c])[None, :]
    b1f = jnp.concatenate([b1_s2, b1_c])[None, :]
    z = jnp.zeros((32, 16), jnp.float32)
    w2s = jnp.concatenate([jnp.concatenate([W2_s1, z], axis=1),
                           jnp.concatenate([z, W2_c], axis=1)], axis=0)
    w2f = jnp.concatenate([jnp.concatenate([W2_s2, z], axis=1),
                           jnp.concatenate([z, W2_c], axis=1)], axis=0)
    b2s = jnp.concatenate([b2_s1, b2_c])[None, :]
    b2f = jnp.concatenate([b2_s2, b2_c])[None, :]

    s_s, s_f = _proj1(x, w1s, w1f)
    emb1, com1, com2, emb2, output, beta3, emb, _, _ = _mega(
        sadj, fadj, s_s, s_f, b1s, b1f, w2s, w2f, b2s, b2f,
        Wp1, bp1[None, :], Wp2[:, 0][None, :], Wm, bm[None, :])

    return (output, beta3[:, :, None], emb1, com1, com2, emb2, emb)


# speedup vs baseline: 1.8823x; 1.0094x over previous
import jax
import jax.numpy as jnp
from jax.experimental import pallas as pl
from jax.experimental.pallas import tpu as pltpu

_N = 10000
_F = 128
_H1 = 64
_H2 = 32
_BR = 200
_BE = 2000

_DN = (((1,), (0,)), ((), ()))


def _proj1_body(x_ref, ws_ref, wf_ref, ss_ref, sf_ref):
    x = x_ref[...]
    ss_ref[...] = jax.lax.dot_general(
        x, ws_ref[...], _DN,
        preferred_element_type=jnp.float32).astype(jnp.bfloat16)
    sf_ref[...] = jax.lax.dot_general(
        x, wf_ref[...], _DN,
        preferred_element_type=jnp.float32).astype(jnp.bfloat16)


def _spmm1_body(sadj_ref, fadj_ref, ss_ref, sf_ref, b1s_ref, b1f_ref,
                w2s_ref, w2f_ref, ts_ref, tf_ref):
    acc_s = jax.lax.dot_general(sadj_ref[...].astype(jnp.bfloat16), ss_ref[...],
                                _DN, preferred_element_type=jnp.float32)
    h_s = jnp.maximum(acc_s + b1s_ref[...], 0.0)
    ts_ref[...] = jax.lax.dot_general(
        h_s, w2s_ref[...], _DN,
        preferred_element_type=jnp.float32).astype(jnp.bfloat16)
    acc_f = jax.lax.dot_general(fadj_ref[...].astype(jnp.bfloat16), sf_ref[...],
                                _DN, preferred_element_type=jnp.float32)
    h_f = jnp.maximum(acc_f + b1f_ref[...], 0.0)
    tf_ref[...] = jax.lax.dot_general(
        h_f, w2f_ref[...], _DN,
        preferred_element_type=jnp.float32).astype(jnp.bfloat16)


def _spmm2_att_body(sadj_ref, fadj_ref, ts_ref, tf_ref, b2s_ref, b2f_ref,
                    wp1_ref, bp1_ref, wp2_ref, wm_ref, bm_ref,
                    emb1_ref, com1_ref, com2_ref, emb2_ref,
                    out_ref, beta_ref, emb_ref):
    o_s = jax.lax.dot_general(sadj_ref[...].astype(jnp.bfloat16), ts_ref[...],
                              _DN, preferred_element_type=jnp.float32) + b2s_ref[...]
    o_f = jax.lax.dot_general(fadj_ref[...].astype(jnp.bfloat16), tf_ref[...],
                              _DN, preferred_element_type=jnp.float32) + b2f_ref[...]
    e1 = o_s[:, :16]
    c1 = o_s[:, 16:]
    e2 = o_f[:, :16]
    c2 = o_f[:, 16:]
    emb1_ref[...] = e1
    com1_ref[...] = c1
    com2_ref[...] = c2
    emb2_ref[...] = e2
    xc = 0.5 * (c1 + c2)

    wp1 = wp1_ref[...]
    bp1 = bp1_ref[...]
    wp2 = wp2_ref[...]

    def att_score(e):
        t = jnp.tanh(jax.lax.dot_general(e, wp1, _DN,
                                         preferred_element_type=jnp.float32)
                     + bp1)
        return jnp.sum(t * wp2, axis=1, keepdims=True)

    w1 = att_score(e1)
    w2 = att_score(e2)
    w3 = att_score(xc)
    m = jnp.maximum(jnp.maximum(w1, w2), w3)
    a1 = jnp.exp(w1 - m)
    a2 = jnp.exp(w2 - m)
    a3 = jnp.exp(w3 - m)
    inv = 1.0 / (a1 + a2 + a3)
    b1 = a1 * inv
    b2 = a2 * inv
    b3 = a3 * inv
    beta_ref[:, 0:1] = b1
    beta_ref[:, 1:2] = b2
    beta_ref[:, 2:3] = b3

    emb = b1 * e1 + b2 * e2 + b3 * xc
    emb_ref[...] = emb

    logits = jax.lax.dot_general(emb, wm_ref[...], _DN,
                                 preferred_element_type=jnp.float32) + bm_ref[...]
    lm = jnp.max(logits, axis=1, keepdims=True)
    lse = jnp.log(jnp.sum(jnp.exp(logits - lm), axis=1, keepdims=True)) + lm
    out_ref[...] = logits - lse


def _proj1(x, w1s, w1f):
    return pl.pallas_call(
        _proj1_body,
        grid=(_N // _BE,),
        in_specs=[
            pl.BlockSpec((_BE, _F), lambda i: (i, 0)),
            pl.BlockSpec((_F, _H1), lambda i: (0, 0)),
            pl.BlockSpec((_F, _H1), lambda i: (0, 0)),
        ],
        out_specs=[
            pl.BlockSpec((_BE, _H1), lambda i: (i, 0)),
            pl.BlockSpec((_BE, _H1), lambda i: (i, 0)),
        ],
        out_shape=[
            jax.ShapeDtypeStruct((_N, _H1), jnp.bfloat16),
            jax.ShapeDtypeStruct((_N, _H1), jnp.bfloat16),
        ],
        compiler_params=pltpu.CompilerParams(
            dimension_semantics=("parallel",)),
    )(x, w1s, w1f)


def _spmm1(sadj, fadj, s_s, s_f, b1s, b1f, w2s, w2f):
    return pl.pallas_call(
        _spmm1_body,
        grid=(_N // _BR,),
        in_specs=[
            pl.BlockSpec((_BR, _N), lambda i: (i, 0)),
            pl.BlockSpec((_BR, _N), lambda i: (i, 0)),
            pl.BlockSpec((_N, _H1), lambda i: (0, 0)),
            pl.BlockSpec((_N, _H1), lambda i: (0, 0)),
            pl.BlockSpec((1, _H1), lambda i: (0, 0)),
            pl.BlockSpec((1, _H1), lambda i: (0, 0)),
            pl.BlockSpec((_H1, _H2), lambda i: (0, 0)),
            pl.BlockSpec((_H1, _H2), lambda i: (0, 0)),
        ],
        out_specs=[
            pl.BlockSpec((_BR, _H2), lambda i: (i, 0)),
            pl.BlockSpec((_BR, _H2), lambda i: (i, 0)),
        ],
        out_shape=[
            jax.ShapeDtypeStruct((_N, _H2), jnp.bfloat16),
            jax.ShapeDtypeStruct((_N, _H2), jnp.bfloat16),
        ],
        compiler_params=pltpu.CompilerParams(
            dimension_semantics=("parallel",)),
    )(sadj, fadj, s_s, s_f, b1s, b1f, w2s, w2f)


def _spmm2_att(sadj, fadj, t_s, t_f, b2s, b2f, wp1, bp1, wp2, wm, bm):
    return pl.pallas_call(
        _spmm2_att_body,
        grid=(_N // _BR,),
        in_specs=[
            pl.BlockSpec((_BR, _N), lambda i: (i, 0)),
            pl.BlockSpec((_BR, _N), lambda i: (i, 0)),
            pl.BlockSpec((_N, _H2), lambda i: (0, 0)),
            pl.BlockSpec((_N, _H2), lambda i: (0, 0)),
            pl.BlockSpec((1, _H2), lambda i: (0, 0)),
            pl.BlockSpec((1, _H2), lambda i: (0, 0)),
            pl.BlockSpec((16, 16), lambda i: (0, 0)),
            pl.BlockSpec((1, 16), lambda i: (0, 0)),
            pl.BlockSpec((1, 16), lambda i: (0, 0)),
            pl.BlockSpec((16, 16), lambda i: (0, 0)),
            pl.BlockSpec((1, 16), lambda i: (0, 0)),
        ],
        out_specs=[
            pl.BlockSpec((_BR, 16), lambda i: (i, 0)),
            pl.BlockSpec((_BR, 16), lambda i: (i, 0)),
            pl.BlockSpec((_BR, 16), lambda i: (i, 0)),
            pl.BlockSpec((_BR, 16), lambda i: (i, 0)),
            pl.BlockSpec((_BR, 16), lambda i: (i, 0)),
            pl.BlockSpec((_BR, 3), lambda i: (i, 0)),
            pl.BlockSpec((_BR, 16), lambda i: (i, 0)),
        ],
        out_shape=[
            jax.ShapeDtypeStruct((_N, 16), jnp.float32),
            jax.ShapeDtypeStruct((_N, 16), jnp.float32),
            jax.ShapeDtypeStruct((_N, 16), jnp.float32),
            jax.ShapeDtypeStruct((_N, 16), jnp.float32),
            jax.ShapeDtypeStruct((_N, 16), jnp.float32),
            jax.ShapeDtypeStruct((_N, 3), jnp.float32),
            jax.ShapeDtypeStruct((_N, 16), jnp.float32),
        ],
        compiler_params=pltpu.CompilerParams(
            dimension_semantics=("parallel",)),
    )(sadj, fadj, t_s, t_f, b2s, b2f, wp1, bp1, wp2, wm, bm)


def kernel(x, sadj, fadj, W1_s1, b1_s1, W2_s1, b2_s1, W1_s2, b1_s2, W2_s2,
           b2_s2, W1_c, b1_c, W2_c, b2_c, Wp1, bp1, Wp2, Wm, bm):
    w1s = jnp.concatenate([W1_s1, W1_c], axis=1)
    w1f = jnp.concatenate([W1_s2, W1_c], axis=1)
    b1s = jnp.concatenate([b1_s1, b1_c])[None, :]
    b1f = jnp.concatenate([b1_s2, b1_c])[None, :]
    z = jnp.zeros((32, 16), jnp.float32)
    w2s = jnp.concatenate([jnp.concatenate([W2_s1, z], axis=1),
                           jnp.concatenate([z, W2_c], axis=1)], axis=0)
    w2f = jnp.concatenate([jnp.concatenate([W2_s2, z], axis=1),
                           jnp.concatenate([z, W2_c], axis=1)], axis=0)
    b2s = jnp.concatenate([b2_s1, b2_c])[None, :]
    b2f = jnp.concatenate([b2_s2, b2_c])[None, :]

    s_s, s_f = _proj1(x, w1s, w1f)
    t_s, t_f = _spmm1(sadj, fadj, s_s, s_f, b1s, b1f, w2s, w2f)
    emb1, com1, com2, emb2, output, beta3, emb = _spmm2_att(
        sadj, fadj, t_s, t_f, b2s, b2f, Wp1, bp1[None, :],
        Wp2[:, 0][None, :], Wm, bm[None, :])

    return (output, beta3[:, :, None], emb1, com1, com2, emb2, emb)
